# Initial kernel scaffold; baseline (speedup 1.0000x reference)
import jax
import jax.numpy as jnp
from jax.experimental import pallas as pl


def kernel(user, item, rating, embed_user_GMF, embed_item_GMF, embed_user_MLP, embed_item_MLP, W1, b1, W2, b2, W3, b3, Wp, bp):
    raise NotImplementedError("write your pallas kernel here")


# speedup vs baseline: 3.0964x; 3.0964x over previous
import functools

import jax
import jax.numpy as jnp
from jax import lax
from jax.experimental import pallas as pl
from jax.experimental.pallas import tpu as pltpu
from jax.experimental.pallas import tpu_sc as plsc

B = 16384
D = 64
DM = 256

NC = 2
NS = 16
NW = NC * NS
BPW = B // NW
CH = 128
NCHUNK = BPW // CH


@functools.lru_cache(maxsize=None)
def _get_sc_gather():
    mesh = plsc.VectorSubcoreMesh(core_axis_name="c", subcore_axis_name="s",
                                  num_cores=NC, num_subcores=NS)

    @functools.partial(
        pl.kernel,
        mesh=mesh,
        out_type=[
            jax.ShapeDtypeStruct((B, 2 * D), jnp.float32),
            jax.ShapeDtypeStruct((B, 2 * D), jnp.float32),
            jax.ShapeDtypeStruct((B, DM), jnp.float32),
            jax.ShapeDtypeStruct((B, DM), jnp.float32),
        ],
        scratch_types=[
            pltpu.VMEM((CH,), jnp.int32),
            pltpu.VMEM((CH,), jnp.int32),
            pltpu.VMEM((CH,), jnp.int32),
            pltpu.VMEM((CH,), jnp.int32),
            pltpu.VMEM((CH, 2 * D), jnp.float32),
            pltpu.VMEM((CH, 2 * D), jnp.float32),
            pltpu.VMEM((CH, DM), jnp.float32),
            pltpu.VMEM((CH, DM), jnp.float32),
            pltpu.SemaphoreType.DMA,
        ],
    )
    def _sc_gather(user_hbm, item_hbm, userh_hbm, itemh_hbm,
                   ug_t, ig_t, um_t, im_t,
                   ug_out, ig_out, um_out, im_out,
                   idxu_v, idxi_v, idxuh_v, idxih_v,
                   ug_v, ig_v, um_v, im_v, sem):
        wid = lax.axis_index("s") * NC + lax.axis_index("c")
        base = wid * BPW
        for c in range(NCHUNK):
            off = base + c * CH
            pltpu.sync_copy(user_hbm.at[pl.ds(off, CH)], idxu_v)
            pltpu.sync_copy(item_hbm.at[pl.ds(off, CH)], idxi_v)
            pltpu.sync_copy(userh_hbm.at[pl.ds(off, CH)], idxuh_v)
            pltpu.sync_copy(itemh_hbm.at[pl.ds(off, CH)], idxih_v)
            cps = [
                pltpu.async_copy(ug_t.at[idxuh_v], ug_v, sem),
                pltpu.async_copy(ig_t.at[idxih_v], ig_v, sem),
                pltpu.async_copy(um_t.at[idxu_v], um_v, sem),
                pltpu.async_copy(im_t.at[idxi_v], im_v, sem),
            ]
            for cp in cps:
                cp.wait()
            pltpu.sync_copy(ug_v, ug_out.at[pl.ds(off, CH)])
            pltpu.sync_copy(ig_v, ig_out.at[pl.ds(off, CH)])
            pltpu.sync_copy(um_v, um_out.at[pl.ds(off, CH)])
            pltpu.sync_copy(im_v, im_out.at[pl.ds(off, CH)])

    return _sc_gather


BT = 2048


def _mlp_body(u, it, ugr, igr, um, im, w1a, w1b, b1, w2t, b2, w3t, b3,
              wpg, wpm, bp, out):
    h = jnp.dot(um[...], w1a[...], preferred_element_type=jnp.float32)
    h = h + jnp.dot(im[...], w1b[...], preferred_element_type=jnp.float32)
    h = jnp.maximum(h + b1[...], 0.0)
    h = jnp.maximum(jnp.dot(h, w2t[...], preferred_element_type=jnp.float32)
                    + b2[...], 0.0)
    m = jnp.maximum(jnp.dot(h, w3t[...], preferred_element_type=jnp.float32)
                    + b3[...], 0.0)
    ug = jnp.where((u[...] & 1) == 1, ugr[...][:, D:], ugr[...][:, :D])
    ig = jnp.where((it[...] & 1) == 1, igr[...][:, D:], igr[...][:, :D])
    g = ug * ig
    out[...] = (jnp.sum(g * wpg[...], axis=1, keepdims=True)
                + jnp.sum(m * wpm[...], axis=1, keepdims=True) + bp[...])


def _tc_mlp(u, it, ugr, igr, um, im, w1a, w1b, b1, w2t, b2, w3t, b3,
            wpg, wpm, bp):
    full = lambda shape: pl.BlockSpec(shape, lambda i: (0,) * len(shape))
    return pl.pallas_call(
        _mlp_body,
        grid=(B // BT,),
        in_specs=[
            pl.BlockSpec((BT, 1), lambda i: (i, 0)),
            pl.BlockSpec((BT, 1), lambda i: (i, 0)),
            pl.BlockSpec((BT, 2 * D), lambda i: (i, 0)),
            pl.BlockSpec((BT, 2 * D), lambda i: (i, 0)),
            pl.BlockSpec((BT, DM), lambda i: (i, 0)),
            pl.BlockSpec((BT, DM), lambda i: (i, 0)),
            full((DM, DM)),
            full((DM, DM)),
            full((1, DM)),
            full((DM, 128)),
            full((1, 128)),
            full((128, D)),
            full((1, D)),
            full((1, D)),
            full((1, D)),
            full((1, 1)),
        ],
        out_specs=pl.BlockSpec((BT, 1), lambda i: (i, 0)),
        out_shape=jax.ShapeDtypeStruct((B, 1), jnp.float32),
    )(u, it, ugr, igr, um, im, w1a, w1b, b1, w2t, b2, w3t, b3, wpg, wpm, bp)


def kernel(user, item, rating, embed_user_GMF, embed_item_GMF,
           embed_user_MLP, embed_item_MLP, W1, b1, W2, b2, W3, b3, Wp, bp):
    user = user.astype(jnp.int32)
    item = item.astype(jnp.int32)
    ug_t = embed_user_GMF.reshape(-1, 2 * D)
    ig_t = embed_item_GMF.reshape(-1, 2 * D)
    ugr, igr, um, im = _get_sc_gather()(user, item, user >> 1, item >> 1,
                                        ug_t, ig_t, embed_user_MLP,
                                        embed_item_MLP)
    w1t = W1.T
    w1a = w1t[:DM]
    w1b = w1t[DM:]
    out = _tc_mlp(user.reshape(B, 1), item.reshape(B, 1), ugr, igr, um, im,
                  w1a, w1b, b1.reshape(1, -1), W2.T, b2.reshape(1, -1),
                  W3.T, b3.reshape(1, -1), Wp[:, :D].reshape(1, D),
                  Wp[:, D:].reshape(1, D), bp.reshape(1, 1))
    return (out, rating)
